# baseline (speedup 1.0000x reference)
import functools

import jax
import jax.numpy as jnp
from jax import lax
from jax.experimental import pallas as pl
from jax.experimental.pallas import tpu as pltpu
from jax.experimental.pallas import tpu_sc as plsc

N_DAYS = 365
N_STEPS = 48
N_IDS = 1024
BATCH = 1024

NC = 2
NS = 16
NW = NC * NS
BPW = BATCH // NW


def _decode_body(cos_ref, sin_ref, idx_ref):
    two_pi = 2.0 * jnp.pi
    ang = jnp.arctan2(sin_ref[...], cos_ref[...])
    doy = jnp.round(jnp.mod(ang, two_pi) / two_pi * 365.0)
    pos = (lax.broadcasted_iota(jnp.int32, (8, BATCH // 8), 0) * (BATCH // 8)
           + lax.broadcasted_iota(jnp.int32, (8, BATCH // 8), 1))
    idx_ref[...] = (doy.astype(jnp.int32) - 1) * BATCH + pos


def _decode_idx(x2):
    m = x2.reshape(BATCH, 2)
    cos8 = m[:, 0].reshape(8, BATCH // 8)
    sin8 = m[:, 1].reshape(8, BATCH // 8)
    idx8 = pl.pallas_call(
        _decode_body,
        out_shape=jax.ShapeDtypeStruct((8, BATCH // 8), jnp.int32),
    )(cos8, sin8)
    return idx8.reshape(BATCH)


def _gather_body(lut_hbm, packed_hbm, out_hbm,
                 packed_v, buf0, buf1, gsem0, gsem1, wsem0, wsem1):
    wid = lax.axis_index("s") * NC + lax.axis_index("c")
    base = wid * BPW
    pltpu.sync_copy(packed_hbm.at[pl.ds(base, BPW)], packed_v)

    packed_c = [packed_v[pl.ds(g * 16, 16)] for g in range(BPW // 16)]

    def at(c, j):
        return c[j // 16][j % 16]

    bufs = (buf0, buf1)
    gsems = (gsem0, gsem1)
    wsems = (wsem0, wsem1)

    def wait_write(j):
        pltpu.make_async_copy(
            bufs[0], out_hbm.at[pl.ds(0, 1)], wsems[j % 2]).wait()

    d_prev = lax.shift_right_logical(at(packed_c, 0), 10) - 1
    run_cnt = jnp.int32(0)
    for j in range(BPW):
        if j >= 2:
            wait_write(j - 2)
        p = at(packed_c, j)
        d = lax.shift_right_logical(p, 10)
        r = lax.bitwise_and(p, BATCH - 1)
        new_run = d != d_prev
        d_prev = d
        run_cnt = run_cnt + new_run.astype(jnp.int32)
        sel = lax.rem(run_cnt - 1, 2)
        for s in (0, 1):
            @pl.when(new_run & (sel == s))
            def _(s=s):
                pltpu.async_copy(lut_hbm.at[pl.ds(d, 1)], bufs[s], gsems[s])

            @pl.when(new_run & (sel == s))
            def _(s=s):
                pltpu.make_async_copy(
                    lut_hbm.at[pl.ds(d, 1)], bufs[s], gsems[s]).wait()

            @pl.when(sel == s)
            def _(s=s):
                pltpu.async_copy(
                    bufs[s], out_hbm.at[pl.ds(r, 1)], wsems[j % 2])
    wait_write(BPW - 2)
    wait_write(BPW - 1)


_sc_gather = functools.partial(
    pl.kernel,
    out_type=jax.ShapeDtypeStruct((BATCH, N_STEPS, N_IDS), jnp.float32),
    mesh=plsc.VectorSubcoreMesh(core_axis_name="c", subcore_axis_name="s",
                                num_cores=NC, num_subcores=NS),
    scratch_types=[
        pltpu.VMEM((BPW,), jnp.int32),
        *[pltpu.VMEM((1, N_STEPS, N_IDS), jnp.float32) for _ in range(2)],
        *[pltpu.SemaphoreType.DMA for _ in range(4)],
    ],
    compiler_params=pltpu.CompilerParams(use_tc_tiling_on_sc=True),
)(_gather_body)


def kernel(x1, x2, lut):
    del x1
    packed = _decode_idx(x2)
    spacked = -lax.top_k(-packed, BATCH)[0]
    return _sc_gather(lut, spacked)


# speedup vs baseline: 1.0016x; 1.0016x over previous
import functools

import jax
import jax.numpy as jnp
from jax import lax
from jax.experimental import pallas as pl
from jax.experimental.pallas import tpu as pltpu
from jax.experimental.pallas import tpu_sc as plsc

N_DAYS = 365
N_STEPS = 48
N_IDS = 1024
BATCH = 1024

NC = 2
NS = 16
NW = NC * NS
BPW = BATCH // NW


def _decode_body(cos_ref, sin_ref, idx_ref):
    two_pi = 2.0 * jnp.pi
    ang = jnp.arctan2(sin_ref[...], cos_ref[...])
    doy = jnp.round(jnp.mod(ang, two_pi) / two_pi * 365.0)
    pos = (lax.broadcasted_iota(jnp.int32, (8, BATCH // 8), 0) * (BATCH // 8)
           + lax.broadcasted_iota(jnp.int32, (8, BATCH // 8), 1))
    idx_ref[...] = (doy.astype(jnp.int32) - 1) * BATCH + pos


def _decode_idx(x2):
    m = x2.reshape(BATCH, 2)
    cos8 = m[:, 0].reshape(8, BATCH // 8)
    sin8 = m[:, 1].reshape(8, BATCH // 8)
    idx8 = pl.pallas_call(
        _decode_body,
        out_shape=jax.ShapeDtypeStruct((8, BATCH // 8), jnp.int32),
    )(cos8, sin8)
    return idx8.reshape(BATCH)


def _gather_body(lut_hbm, packed_hbm, out_hbm,
                 packed_v, buf0, buf1, gsem0, gsem1, wsem0, wsem1):
    wid = lax.axis_index("s") * NC + lax.axis_index("c")
    base = wid * BPW
    pltpu.sync_copy(packed_hbm.at[pl.ds(base, BPW)], packed_v)

    packed_c = [packed_v[pl.ds(g * 16, 16)] for g in range(BPW // 16)]

    def at(c, j):
        return c[j // 16][j % 16]

    bufs = (buf0, buf1)
    gsems = (gsem0, gsem1)
    wsems = (wsem0, wsem1)

    def wait_write(j):
        pltpu.make_async_copy(
            bufs[0], out_hbm.at[pl.ds(0, 1)], wsems[j % 2]).wait()

    d_prev = lax.shift_right_logical(at(packed_c, 0), 10) - 1
    run_cnt = jnp.int32(0)
    for j in range(BPW):
        if j >= 2:
            wait_write(j - 2)
        p = at(packed_c, j)
        d = lax.shift_right_logical(p, 10)
        r = lax.bitwise_and(p, BATCH - 1)
        new_run = d != d_prev
        d_prev = d
        run_cnt = run_cnt + new_run.astype(jnp.int32)
        sel = lax.rem(run_cnt - 1, 2)
        for s in (0, 1):
            @pl.when(new_run & (sel == s))
            def _(s=s):
                pltpu.async_copy(lut_hbm.at[pl.ds(d, 1)], bufs[s], gsems[s])

            @pl.when(new_run & (sel == s))
            def _(s=s):
                pltpu.make_async_copy(
                    lut_hbm.at[pl.ds(d, 1)], bufs[s], gsems[s]).wait()

            @pl.when(sel == s)
            def _(s=s):
                pltpu.async_copy(
                    bufs[s], out_hbm.at[pl.ds(r, 1)], wsems[j % 2])
    wait_write(BPW - 2)
    wait_write(BPW - 1)


_sc_gather = functools.partial(
    pl.kernel,
    out_type=jax.ShapeDtypeStruct((BATCH, N_STEPS, N_IDS), jnp.float32),
    mesh=plsc.VectorSubcoreMesh(core_axis_name="c", subcore_axis_name="s",
                                num_cores=NC, num_subcores=NS),
    scratch_types=[
        pltpu.VMEM((BPW,), jnp.int32),
        *[pltpu.VMEM((1, N_STEPS, N_IDS), jnp.float32) for _ in range(2)],
        *[pltpu.SemaphoreType.DMA for _ in range(4)],
    ],
    compiler_params=pltpu.CompilerParams(use_tc_tiling_on_sc=True),
)(_gather_body)


def kernel(x1, x2, lut):
    del x1
    packed = _decode_idx(x2)
    return _sc_gather(lut, lax.sort(packed))
